# Initial kernel scaffold; baseline (speedup 1.0000x reference)
import jax
import jax.numpy as jnp
from jax.experimental import pallas as pl


def kernel(x, pos, batch, params):
    raise NotImplementedError("write your pallas kernel here")


# speedup vs baseline: 5.8883x; 5.8883x over previous
import functools

import jax
import jax.numpy as jnp
from jax import lax
from jax.experimental import pallas as pl
from jax.experimental.pallas import tpu as pltpu
from jax.experimental.pallas import tpu_sc as plsc

B = 4
P = 1024
K = 30
N = B * P
E = N * K
BIG = 1e30


def _pad_cols(a, m=8):
    d = a.shape[-1]
    pad = (-d) % m
    if pad == 0:
        return a
    return jnp.concatenate([a, jnp.zeros(a.shape[:-1] + (pad,), a.dtype)], axis=-1)


def _pad_rows(a, m=8):
    d = a.shape[0]
    pad = (-d) % m
    if pad == 0:
        return a
    return jnp.concatenate([a, jnp.zeros((pad,) + a.shape[1:], a.dtype)], axis=0)


def _knn_body(f_ref, o_ref):
    b = pl.program_id(0)
    fb = f_ref[0]
    sq = jnp.sum(fb * fb, axis=1, keepdims=True)
    cross = lax.dot_general(fb, fb, (((1,), (1,)), ((), ())),
                            preferred_element_type=jnp.float32)
    d2 = sq + sq.T - 2.0 * cross
    col = lax.broadcasted_iota(jnp.int32, (P, P), 1)
    row = lax.broadcasted_iota(jnp.int32, (P, P), 0)
    d2 = jnp.where(col == row, 1e10, d2)
    cols = []
    for _ in range(K):
        m = jnp.min(d2, axis=1, keepdims=True)
        amin = jnp.min(jnp.where(d2 <= m, col, P), axis=1)
        cols.append(amin[:, None])
        d2 = jnp.where(col == amin[:, None], BIG, d2)
    cols.append(jnp.zeros((P, 2), jnp.int32))
    nbr = jnp.concatenate(cols, axis=1)
    o_ref[0] = nbr + b * P


def _knn(f):
    D = f.shape[-1]
    out = pl.pallas_call(
        _knn_body,
        grid=(B,),
        in_specs=[pl.BlockSpec((1, P, D), lambda b: (b, 0, 0))],
        out_specs=pl.BlockSpec((1, P, 32), lambda b: (b, 0, 0)),
        out_shape=jax.ShapeDtypeStruct((B, P, 32), jnp.int32),
    )(f)
    return out[:, :, :K]


def _matmul_body(a_ref, w_ref, b_ref, o_ref):
    o_ref[...] = (
        jnp.dot(a_ref[...], w_ref[...], preferred_element_type=jnp.float32)
        + b_ref[...]
    )


def _matmul(a, w, bias):
    n, din = a.shape
    dout = w.shape[1]
    blk = 512
    return pl.pallas_call(
        _matmul_body,
        grid=(n // blk,),
        in_specs=[
            pl.BlockSpec((blk, din), lambda j: (j, 0)),
            pl.BlockSpec((din, dout), lambda j: (0, 0)),
            pl.BlockSpec((1, dout), lambda j: (0, 0)),
        ],
        out_specs=pl.BlockSpec((blk, dout), lambda j: (j, 0)),
        out_shape=jax.ShapeDtypeStruct((n, dout), jnp.float32),
    )(a, w, bias)


def _sc_gather(table, idx2d, F):
    nw = 32
    rpw = idx2d.shape[1]

    mesh = plsc.VectorSubcoreMesh(core_axis_name="c", subcore_axis_name="s")

    @functools.partial(
        pl.kernel,
        mesh=mesh,
        compiler_params=pltpu.CompilerParams(use_tc_tiling_on_sc=False),
        out_type=jax.ShapeDtypeStruct((E, F), jnp.float32),
        scratch_types=[
            pltpu.VMEM((rpw, 128), jnp.int32),
            pltpu.VMEM((128, F), jnp.float32),
            pltpu.SemaphoreType.DMA,
        ],
    )
    def gk(idx_hbm, table_hbm, out_hbm, idx_v, rows_v, sem):
        c = lax.axis_index("c")
        s = lax.axis_index("s")
        wid = s * 2 + c
        pltpu.sync_copy(idx_hbm.at[wid], idx_v)

        def body(i, carry):
            pltpu.async_copy(table_hbm.at[idx_v.at[i]], rows_v, sem).wait()
            pltpu.sync_copy(
                rows_v, out_hbm.at[pl.ds((wid * rpw + i) * 128, 128)]
            )
            return carry

        lax.fori_loop(0, rpw, body, 0)

    return gk(idx2d, table)


def _edge_body(w_ref, vg_ref, W2_ref, b2_ref, g_ref, bt_ref, o_ref, st_ref):
    p = pl.program_id(0)
    j = pl.program_id(1)
    h1 = vg_ref[...] + w_ref[...][None]

    @pl.when((p == 0) & (j == 0))
    def _():
        st_ref[...] = jnp.zeros_like(st_ref)

    @pl.when(p == 0)
    def _():
        s1 = jnp.sum(jnp.sum(h1, axis=0), axis=0)
        s2 = jnp.sum(jnp.sum(h1 * h1, axis=0), axis=0)
        st_ref[0:1, :] += s1[None]
        st_ref[1:2, :] += s2[None]
        o_ref[...] = jnp.zeros_like(o_ref)

    @pl.when(p == 1)
    def _():
        cnt = float(E)
        mu = st_ref[0:1, :] / cnt
        var = st_ref[1:2, :] / cnt - mu * mu
        sc = g_ref[...] * lax.rsqrt(var + 1e-5)
        hn = (h1 - mu[None]) * sc[None] + bt_ref[...][None]
        hr = jnp.maximum(hn, 0.0)
        kk, r, f1 = hr.shape
        h2 = jnp.dot(
            jnp.reshape(hr, (kk * r, f1)),
            W2_ref[...],
            preferred_element_type=jnp.float32,
        ) + b2_ref[...]
        f2 = h2.shape[1]
        o_ref[...] = jnp.max(jnp.reshape(h2, (kk, r, f2)), axis=0)


def _edge_stage(w, vg3, W2, b2, gamma, beta):
    F1 = w.shape[1]
    F2 = W2.shape[1]
    nblk = 16
    R = N // nblk
    return pl.pallas_call(
        _edge_body,
        grid=(2, nblk),
        in_specs=[
            pl.BlockSpec((R, F1), lambda p, j: (j, 0)),
            pl.BlockSpec((K, R, F1), lambda p, j: (0, j, 0)),
            pl.BlockSpec((F1, F2), lambda p, j: (0, 0)),
            pl.BlockSpec((1, F2), lambda p, j: (0, 0)),
            pl.BlockSpec((1, F1), lambda p, j: (0, 0)),
            pl.BlockSpec((1, F1), lambda p, j: (0, 0)),
        ],
        out_specs=pl.BlockSpec((R, F2), lambda p, j: (j, 0)),
        out_shape=jax.ShapeDtypeStruct((N, F2), jnp.float32),
        scratch_shapes=[pltpu.VMEM((2, F1), jnp.float32)],
    )(w, vg3, W2, b2[None], gamma[None], beta[None])


def _edge_conv(feat, nbr, layers):
    (W1, b1, gamma, beta), (W2, b2) = layers
    d = feat.shape[1]
    Wt, Wb = W1[:d], W1[d:]
    Wd = jnp.concatenate([Wt - Wb, Wb], axis=1)
    F1 = W2.shape[0]
    bias = jnp.concatenate([b1, jnp.zeros_like(b1)])[None]
    uv = _matmul(_pad_cols(feat), _pad_rows(Wd), bias)
    w = uv[:, :F1]
    v = uv[:, F1:]
    idx = jnp.reshape(jnp.transpose(jnp.reshape(nbr, (N, K))), (32, E // (32 * 128), 128))
    vg = _sc_gather(v, idx, F1)
    vg3 = jnp.reshape(vg, (K, N, F1))
    return _edge_stage(w, vg3, W2, b2, gamma, beta)


def _fe_body(a_ref, W1_ref, b1_ref, g_ref, bt_ref, W2_ref, b2_ref,
             o_ref, st_ref, mx_ref):
    p = pl.program_id(0)
    j = pl.program_id(1)
    nblk = pl.num_programs(1)
    t = jnp.dot(a_ref[...], W1_ref[...],
                preferred_element_type=jnp.float32) + b1_ref[...]

    @pl.when((p == 0) & (j == 0))
    def _():
        st_ref[...] = jnp.zeros_like(st_ref)

    @pl.when(p == 0)
    def _():
        st_ref[0:1, :] += jnp.sum(t, axis=0)[None]
        st_ref[1:2, :] += jnp.sum(t * t, axis=0)[None]
        o_ref[...] = jnp.zeros_like(o_ref)

    @pl.when(p == 1)
    def _():
        cnt = float(N)
        mu = st_ref[0:1, :] / cnt
        var = st_ref[1:2, :] / cnt - mu * mu
        sc = g_ref[...] * lax.rsqrt(var + 1e-5)
        hr = jnp.maximum((t - mu) * sc + bt_ref[...], 0.0)
        x5 = jnp.dot(hr, W2_ref[...],
                     preferred_element_type=jnp.float32) + b2_ref[...]
        bm = jnp.max(x5, axis=0)[None]

        @pl.when(j == 0)
        def _():
            mx_ref[...] = jnp.full_like(mx_ref, -BIG)

        mx_ref[...] = jnp.maximum(mx_ref[...], bm)

        @pl.when(j == nblk - 1)
        def _():
            o_ref[...] = mx_ref[...]


def _feature_extract(l4p, fe):
    (W1, b1, gamma, beta), (W2, b2) = fe
    W1p = _pad_rows(W1)
    din = l4p.shape[1]
    H = W1.shape[1]
    nblk = 8
    R = N // nblk
    return pl.pallas_call(
        _fe_body,
        grid=(2, nblk),
        in_specs=[
            pl.BlockSpec((R, din), lambda p, j: (j, 0)),
            pl.BlockSpec((din, H), lambda p, j: (0, 0)),
            pl.BlockSpec((1, H), lambda p, j: (0, 0)),
            pl.BlockSpec((1, H), lambda p, j: (0, 0)),
            pl.BlockSpec((1, H), lambda p, j: (0, 0)),
            pl.BlockSpec((H, H), lambda p, j: (0, 0)),
            pl.BlockSpec((1, H), lambda p, j: (0, 0)),
        ],
        out_specs=pl.BlockSpec((1, H), lambda p, j: (0, 0)),
        out_shape=jax.ShapeDtypeStruct((1, H), jnp.float32),
        scratch_shapes=[
            pltpu.VMEM((2, H), jnp.float32),
            pltpu.VMEM((1, H), jnp.float32),
        ],
    )(l4p, W1p, b1[None], gamma[None], beta[None], W2, b2[None])


def _head_body(a_ref, gf_ref, W0t_ref, W0b_ref, b0_ref, W1_ref, b1_ref,
               W2_ref, b2_ref, W3_ref, b3_ref, o_ref):
    gbias = jnp.dot(gf_ref[...], W0b_ref[...],
                    preferred_element_type=jnp.float32) + b0_ref[...]
    h = jnp.dot(a_ref[...], W0t_ref[...],
                preferred_element_type=jnp.float32) + gbias
    h = jnp.maximum(h, 0.0)
    h = jnp.maximum(
        jnp.dot(h, W1_ref[...], preferred_element_type=jnp.float32)
        + b1_ref[...], 0.0)
    h = jnp.maximum(
        jnp.dot(h, W2_ref[...], preferred_element_type=jnp.float32)
        + b2_ref[...], 0.0)
    o = jnp.dot(h, W3_ref[...], preferred_element_type=jnp.float32) \
        + b3_ref[...]
    colmask = lax.broadcasted_iota(jnp.int32, o.shape, 1) < 50
    o = jnp.where(colmask, o, -BIG)
    m = jnp.max(o, axis=1, keepdims=True)
    z = jnp.sum(jnp.exp(o - m), axis=1, keepdims=True)
    o_ref[...] = o - m - jnp.log(z)


def _head(l4p, gf, head):
    (W0, b0), (W1, b1), (W2, b2), (W3, b3) = head
    din = l4p.shape[1]
    W0t = _pad_rows(W0[:326])
    W0b = W0[326:]
    W3p = _pad_cols(W3, 64)
    b3p = _pad_cols(b3[None], 64)
    blk = 512
    out = pl.pallas_call(
        _head_body,
        grid=(N // blk,),
        in_specs=[
            pl.BlockSpec((blk, din), lambda j: (j, 0)),
            pl.BlockSpec((1, 1024), lambda j: (0, 0)),
            pl.BlockSpec((din, 256), lambda j: (0, 0)),
            pl.BlockSpec((1024, 256), lambda j: (0, 0)),
            pl.BlockSpec((1, 256), lambda j: (0, 0)),
            pl.BlockSpec((256, 256), lambda j: (0, 0)),
            pl.BlockSpec((1, 256), lambda j: (0, 0)),
            pl.BlockSpec((256, 128), lambda j: (0, 0)),
            pl.BlockSpec((1, 128), lambda j: (0, 0)),
            pl.BlockSpec((128, 64), lambda j: (0, 0)),
            pl.BlockSpec((1, 64), lambda j: (0, 0)),
        ],
        out_specs=pl.BlockSpec((blk, 64), lambda j: (j, 0)),
        out_shape=jax.ShapeDtypeStruct((N, 64), jnp.float32),
    )(l4p, gf, W0t, W0b, b0[None], W1, b1[None], W2, b2[None], W3p, b3p)
    return out[:, :50]


def kernel(x, pos, batch, params):
    x0 = jnp.concatenate([x, pos], axis=1)

    nbr = _knn(_pad_cols(x0).reshape(B, P, 8))
    x1 = _edge_conv(x0, nbr, params["ec1"])

    nbr = _knn(x1.reshape(B, P, 64))
    l1 = jnp.concatenate([x0, x1], axis=1)
    x2 = _edge_conv(l1, nbr, params["ec2"])

    nbr = _knn(x2.reshape(B, P, 64))
    l2 = jnp.concatenate([x0, x1, x2], axis=1)
    x3 = _edge_conv(l2, nbr, params["ec3"])

    l3 = jnp.concatenate([x0, x1, x2, x3], axis=1)
    x4 = _edge_conv(l3, nbr, params["ec4"])

    l4 = jnp.concatenate([x0, x1, x2, x3, x4], axis=1)
    l4p = _pad_cols(l4)
    gf = _feature_extract(l4p, params["fe"])
    return _head(l4p, gf, params["head"])


# speedup vs baseline: 7.0842x; 1.2031x over previous
import functools

import jax
import jax.numpy as jnp
from jax import lax
from jax.experimental import pallas as pl
from jax.experimental.pallas import tpu as pltpu
from jax.experimental.pallas import tpu_sc as plsc

B = 4
P = 1024
K = 30
N = B * P
E = N * K
BIG = 1e30


def _pad_cols(a, m=8):
    d = a.shape[-1]
    pad = (-d) % m
    if pad == 0:
        return a
    return jnp.concatenate([a, jnp.zeros(a.shape[:-1] + (pad,), a.dtype)], axis=-1)


def _pad_rows(a, m=8):
    d = a.shape[0]
    pad = (-d) % m
    if pad == 0:
        return a
    return jnp.concatenate([a, jnp.zeros((pad,) + a.shape[1:], a.dtype)], axis=0)


def _knn_body(f_ref, o_ref):
    b = pl.program_id(0)
    fb = f_ref[0]
    sq = jnp.sum(fb * fb, axis=1, keepdims=True)
    cross = lax.dot_general(fb, fb, (((1,), (1,)), ((), ())),
                            preferred_element_type=jnp.float32)
    d2 = sq + sq.T - 2.0 * cross
    col = lax.broadcasted_iota(jnp.int32, (P, P), 1)
    row = lax.broadcasted_iota(jnp.int32, (P, P), 0)
    d2 = jnp.where(col == row, 1e10, d2)
    colf = col.astype(jnp.float32)
    cols = []
    for _ in range(K):
        m = jnp.min(d2, axis=1, keepdims=True)
        eq = d2 <= m
        amin = jnp.min(jnp.where(eq, colf, 1e9), axis=1)
        cols.append(amin[:, None])
        d2 = jnp.where(eq, BIG, d2)
    cols.append(jnp.zeros((P, 2), jnp.float32))
    nbr = jnp.concatenate(cols, axis=1).astype(jnp.int32)
    o_ref[0] = nbr + b * P


def _knn(f):
    D = f.shape[-1]
    out = pl.pallas_call(
        _knn_body,
        grid=(B,),
        in_specs=[pl.BlockSpec((1, P, D), lambda b: (b, 0, 0))],
        out_specs=pl.BlockSpec((1, P, 32), lambda b: (b, 0, 0)),
        out_shape=jax.ShapeDtypeStruct((B, P, 32), jnp.int32),
    )(f)
    return out[:, :, :K]


def _matmul_body(a_ref, w_ref, b_ref, o_ref):
    o_ref[...] = (
        jnp.dot(a_ref[...], w_ref[...], preferred_element_type=jnp.float32)
        + b_ref[...]
    )


def _matmul(a, w, bias):
    n, din = a.shape
    dout = w.shape[1]
    blk = 512
    return pl.pallas_call(
        _matmul_body,
        grid=(n // blk,),
        in_specs=[
            pl.BlockSpec((blk, din), lambda j: (j, 0)),
            pl.BlockSpec((din, dout), lambda j: (0, 0)),
            pl.BlockSpec((1, dout), lambda j: (0, 0)),
        ],
        out_specs=pl.BlockSpec((blk, dout), lambda j: (j, 0)),
        out_shape=jax.ShapeDtypeStruct((n, dout), jnp.float32),
    )(a, w, bias)


def _sc_gather(table, idx2d, F):
    nw = 32
    rpw = idx2d.shape[1]

    mesh = plsc.VectorSubcoreMesh(core_axis_name="c", subcore_axis_name="s")

    @functools.partial(
        pl.kernel,
        mesh=mesh,
        compiler_params=pltpu.CompilerParams(use_tc_tiling_on_sc=False),
        out_type=jax.ShapeDtypeStruct((E, F), jnp.float32),
        scratch_types=[
            pltpu.VMEM((rpw, 128), jnp.int32),
            pltpu.VMEM((128, F), jnp.float32),
            pltpu.VMEM((128, F), jnp.float32),
            pltpu.SemaphoreType.DMA,
            pltpu.SemaphoreType.DMA,
            pltpu.SemaphoreType.DMA,
            pltpu.SemaphoreType.DMA,
        ],
    )
    def gk(idx_hbm, table_hbm, out_hbm, idx_v, rows_a, rows_b, sia, sib,
           soa, sob):
        c = lax.axis_index("c")
        s = lax.axis_index("s")
        wid = s * 2 + c
        pltpu.sync_copy(idx_hbm.at[wid], idx_v)

        rows = (rows_a, rows_b)
        sin = (sia, sib)
        sout = (soa, sob)

        def start_in(j, b):
            return pltpu.async_copy(table_hbm.at[idx_v.at[j]], rows[b], sin[b])

        def start_out(j, b):
            return pltpu.async_copy(
                rows[b], out_hbm.at[pl.ds((wid * rpw + j) * 128, 128)],
                sout[b])

        hin = [None] * rpw
        hout = [None] * rpw
        hin[0] = start_in(0, 0)
        for j in range(rpw):
            b = j % 2
            if j + 1 < rpw:
                if j >= 1:
                    hout[j - 1].wait()
                hin[j + 1] = start_in(j + 1, (j + 1) % 2)
            hin[j].wait()
            hout[j] = start_out(j, b)
        if rpw >= 2:
            hout[rpw - 2].wait()
        hout[rpw - 1].wait()

    return gk(idx2d, table)


def _edge_body(w_ref, vg_ref, W2_ref, b2_ref, g_ref, bt_ref, o_ref, st_ref):
    p = pl.program_id(0)
    j = pl.program_id(1)
    h1 = vg_ref[...] + w_ref[...][None]

    @pl.when((p == 0) & (j == 0))
    def _():
        st_ref[...] = jnp.zeros_like(st_ref)

    @pl.when(p == 0)
    def _():
        s1 = jnp.sum(jnp.sum(h1, axis=0), axis=0)
        s2 = jnp.sum(jnp.sum(h1 * h1, axis=0), axis=0)
        st_ref[0:1, :] += s1[None]
        st_ref[1:2, :] += s2[None]
        o_ref[...] = jnp.zeros_like(o_ref)

    @pl.when(p == 1)
    def _():
        cnt = float(E)
        mu = st_ref[0:1, :] / cnt
        var = st_ref[1:2, :] / cnt - mu * mu
        sc = g_ref[...] * lax.rsqrt(var + 1e-5)
        hn = (h1 - mu[None]) * sc[None] + bt_ref[...][None]
        hr = jnp.maximum(hn, 0.0)
        kk, r, f1 = hr.shape
        h2 = jnp.dot(
            jnp.reshape(hr, (kk * r, f1)),
            W2_ref[...],
            preferred_element_type=jnp.float32,
        ) + b2_ref[...]
        f2 = h2.shape[1]
        o_ref[...] = jnp.max(jnp.reshape(h2, (kk, r, f2)), axis=0)


def _edge_stage(w, vg3, W2, b2, gamma, beta):
    F1 = w.shape[1]
    F2 = W2.shape[1]
    nblk = 16
    R = N // nblk
    return pl.pallas_call(
        _edge_body,
        grid=(2, nblk),
        in_specs=[
            pl.BlockSpec((R, F1), lambda p, j: (j, 0)),
            pl.BlockSpec((K, R, F1), lambda p, j: (0, j, 0)),
            pl.BlockSpec((F1, F2), lambda p, j: (0, 0)),
            pl.BlockSpec((1, F2), lambda p, j: (0, 0)),
            pl.BlockSpec((1, F1), lambda p, j: (0, 0)),
            pl.BlockSpec((1, F1), lambda p, j: (0, 0)),
        ],
        out_specs=pl.BlockSpec((R, F2), lambda p, j: (j, 0)),
        out_shape=jax.ShapeDtypeStruct((N, F2), jnp.float32),
        scratch_shapes=[pltpu.VMEM((2, F1), jnp.float32)],
    )(w, vg3, W2, b2[None], gamma[None], beta[None])


def _edge_conv(feat, nbr, layers):
    (W1, b1, gamma, beta), (W2, b2) = layers
    d = feat.shape[1]
    Wt, Wb = W1[:d], W1[d:]
    Wd = jnp.concatenate([Wt - Wb, Wb], axis=1)
    F1 = W2.shape[0]
    bias = jnp.concatenate([b1, jnp.zeros_like(b1)])[None]
    uv = _matmul(_pad_cols(feat), _pad_rows(Wd), bias)
    w = uv[:, :F1]
    v = uv[:, F1:]
    idx = jnp.reshape(jnp.transpose(jnp.reshape(nbr, (N, K))), (32, E // (32 * 128), 128))
    vg = _sc_gather(v, idx, F1)
    vg3 = jnp.reshape(vg, (K, N, F1))
    return _edge_stage(w, vg3, W2, b2, gamma, beta)


def _fe_body(a_ref, W1_ref, b1_ref, g_ref, bt_ref, W2_ref, b2_ref,
             o_ref, st_ref, mx_ref):
    p = pl.program_id(0)
    j = pl.program_id(1)
    nblk = pl.num_programs(1)
    t = jnp.dot(a_ref[...], W1_ref[...],
                preferred_element_type=jnp.float32) + b1_ref[...]

    @pl.when((p == 0) & (j == 0))
    def _():
        st_ref[...] = jnp.zeros_like(st_ref)

    @pl.when(p == 0)
    def _():
        st_ref[0:1, :] += jnp.sum(t, axis=0)[None]
        st_ref[1:2, :] += jnp.sum(t * t, axis=0)[None]
        o_ref[...] = jnp.zeros_like(o_ref)

    @pl.when(p == 1)
    def _():
        cnt = float(N)
        mu = st_ref[0:1, :] / cnt
        var = st_ref[1:2, :] / cnt - mu * mu
        sc = g_ref[...] * lax.rsqrt(var + 1e-5)
        hr = jnp.maximum((t - mu) * sc + bt_ref[...], 0.0)
        x5 = jnp.dot(hr, W2_ref[...],
                     preferred_element_type=jnp.float32) + b2_ref[...]
        bm = jnp.max(x5, axis=0)[None]

        @pl.when(j == 0)
        def _():
            mx_ref[...] = jnp.full_like(mx_ref, -BIG)

        mx_ref[...] = jnp.maximum(mx_ref[...], bm)

        @pl.when(j == nblk - 1)
        def _():
            o_ref[...] = mx_ref[...]


def _feature_extract(l4p, fe):
    (W1, b1, gamma, beta), (W2, b2) = fe
    W1p = _pad_rows(W1)
    din = l4p.shape[1]
    H = W1.shape[1]
    nblk = 8
    R = N // nblk
    return pl.pallas_call(
        _fe_body,
        grid=(2, nblk),
        in_specs=[
            pl.BlockSpec((R, din), lambda p, j: (j, 0)),
            pl.BlockSpec((din, H), lambda p, j: (0, 0)),
            pl.BlockSpec((1, H), lambda p, j: (0, 0)),
            pl.BlockSpec((1, H), lambda p, j: (0, 0)),
            pl.BlockSpec((1, H), lambda p, j: (0, 0)),
            pl.BlockSpec((H, H), lambda p, j: (0, 0)),
            pl.BlockSpec((1, H), lambda p, j: (0, 0)),
        ],
        out_specs=pl.BlockSpec((1, H), lambda p, j: (0, 0)),
        out_shape=jax.ShapeDtypeStruct((1, H), jnp.float32),
        scratch_shapes=[
            pltpu.VMEM((2, H), jnp.float32),
            pltpu.VMEM((1, H), jnp.float32),
        ],
    )(l4p, W1p, b1[None], gamma[None], beta[None], W2, b2[None])


def _head_body(a_ref, gf_ref, W0t_ref, W0b_ref, b0_ref, W1_ref, b1_ref,
               W2_ref, b2_ref, W3_ref, b3_ref, o_ref):
    gbias = jnp.dot(gf_ref[...], W0b_ref[...],
                    preferred_element_type=jnp.float32) + b0_ref[...]
    h = jnp.dot(a_ref[...], W0t_ref[...],
                preferred_element_type=jnp.float32) + gbias
    h = jnp.maximum(h, 0.0)
    h = jnp.maximum(
        jnp.dot(h, W1_ref[...], preferred_element_type=jnp.float32)
        + b1_ref[...], 0.0)
    h = jnp.maximum(
        jnp.dot(h, W2_ref[...], preferred_element_type=jnp.float32)
        + b2_ref[...], 0.0)
    o = jnp.dot(h, W3_ref[...], preferred_element_type=jnp.float32) \
        + b3_ref[...]
    colmask = lax.broadcasted_iota(jnp.int32, o.shape, 1) < 50
    o = jnp.where(colmask, o, -BIG)
    m = jnp.max(o, axis=1, keepdims=True)
    z = jnp.sum(jnp.exp(o - m), axis=1, keepdims=True)
    o_ref[...] = o - m - jnp.log(z)


def _head(l4p, gf, head):
    (W0, b0), (W1, b1), (W2, b2), (W3, b3) = head
    din = l4p.shape[1]
    W0t = _pad_rows(W0[:326])
    W0b = W0[326:]
    W3p = _pad_cols(W3, 64)
    b3p = _pad_cols(b3[None], 64)
    blk = 512
    out = pl.pallas_call(
        _head_body,
        grid=(N // blk,),
        in_specs=[
            pl.BlockSpec((blk, din), lambda j: (j, 0)),
            pl.BlockSpec((1, 1024), lambda j: (0, 0)),
            pl.BlockSpec((din, 256), lambda j: (0, 0)),
            pl.BlockSpec((1024, 256), lambda j: (0, 0)),
            pl.BlockSpec((1, 256), lambda j: (0, 0)),
            pl.BlockSpec((256, 256), lambda j: (0, 0)),
            pl.BlockSpec((1, 256), lambda j: (0, 0)),
            pl.BlockSpec((256, 128), lambda j: (0, 0)),
            pl.BlockSpec((1, 128), lambda j: (0, 0)),
            pl.BlockSpec((128, 64), lambda j: (0, 0)),
            pl.BlockSpec((1, 64), lambda j: (0, 0)),
        ],
        out_specs=pl.BlockSpec((blk, 64), lambda j: (j, 0)),
        out_shape=jax.ShapeDtypeStruct((N, 64), jnp.float32),
    )(l4p, gf, W0t, W0b, b0[None], W1, b1[None], W2, b2[None], W3p, b3p)
    return out[:, :50]


def kernel(x, pos, batch, params):
    x0 = jnp.concatenate([x, pos], axis=1)

    nbr = _knn(_pad_cols(x0).reshape(B, P, 8))
    x1 = _edge_conv(x0, nbr, params["ec1"])

    nbr = _knn(x1.reshape(B, P, 64))
    l1 = jnp.concatenate([x0, x1], axis=1)
    x2 = _edge_conv(l1, nbr, params["ec2"])

    nbr = _knn(x2.reshape(B, P, 64))
    l2 = jnp.concatenate([x0, x1, x2], axis=1)
    x3 = _edge_conv(l2, nbr, params["ec3"])

    l3 = jnp.concatenate([x0, x1, x2, x3], axis=1)
    x4 = _edge_conv(l3, nbr, params["ec4"])

    l4 = jnp.concatenate([x0, x1, x2, x3, x4], axis=1)
    l4p = _pad_cols(l4)
    gf = _feature_extract(l4p, params["fe"])
    return _head(l4p, gf, params["head"])


# speedup vs baseline: 7.3478x; 1.0372x over previous
import functools

import jax
import jax.numpy as jnp
from jax import lax
from jax.experimental import pallas as pl
from jax.experimental.pallas import tpu as pltpu
from jax.experimental.pallas import tpu_sc as plsc

B = 4
P = 1024
K = 30
N = B * P
E = N * K
BIG = 1e30


def _pad_cols(a, m=8):
    d = a.shape[-1]
    pad = (-d) % m
    if pad == 0:
        return a
    return jnp.concatenate([a, jnp.zeros(a.shape[:-1] + (pad,), a.dtype)], axis=-1)


def _pad_rows(a, m=8):
    d = a.shape[0]
    pad = (-d) % m
    if pad == 0:
        return a
    return jnp.concatenate([a, jnp.zeros((pad,) + a.shape[1:], a.dtype)], axis=0)


def _knn_body(f_ref, o_ref):
    b = pl.program_id(0)
    fb = f_ref[0]
    sq = jnp.sum(fb * fb, axis=1, keepdims=True)
    cross = lax.dot_general(fb, fb, (((1,), (1,)), ((), ())),
                            preferred_element_type=jnp.float32)
    d2 = sq + sq.T - 2.0 * cross
    col = lax.broadcasted_iota(jnp.int32, (P, P), 1)
    row = lax.broadcasted_iota(jnp.int32, (P, P), 0)
    d2 = jnp.where(col == row, 1e10, d2)
    colf = col.astype(jnp.float32)
    cols = []
    for _ in range(K):
        m = jnp.min(d2, axis=1, keepdims=True)
        eq = d2 <= m
        amin = jnp.min(jnp.where(eq, colf, 1e9), axis=1)
        cols.append(amin[:, None])
        d2 = jnp.where(eq, BIG, d2)
    cols.append(jnp.zeros((P, 2), jnp.float32))
    nbr = jnp.concatenate(cols, axis=1).astype(jnp.int32)
    o_ref[0] = nbr + b * P


def _knn(f):
    D = f.shape[-1]
    out = pl.pallas_call(
        _knn_body,
        grid=(B,),
        in_specs=[pl.BlockSpec((1, P, D), lambda b: (b, 0, 0))],
        out_specs=pl.BlockSpec((1, P, 32), lambda b: (b, 0, 0)),
        out_shape=jax.ShapeDtypeStruct((B, P, 32), jnp.int32),
    )(f)
    return out[:, :, :K]


def _matmul_body(a_ref, w_ref, b_ref, o_ref):
    o_ref[...] = (
        jnp.dot(a_ref[...], w_ref[...], preferred_element_type=jnp.float32)
        + b_ref[...]
    )


def _matmul(a, w, bias):
    n, din = a.shape
    dout = w.shape[1]
    blk = 512
    return pl.pallas_call(
        _matmul_body,
        grid=(n // blk,),
        in_specs=[
            pl.BlockSpec((blk, din), lambda j: (j, 0)),
            pl.BlockSpec((din, dout), lambda j: (0, 0)),
            pl.BlockSpec((1, dout), lambda j: (0, 0)),
        ],
        out_specs=pl.BlockSpec((blk, dout), lambda j: (j, 0)),
        out_shape=jax.ShapeDtypeStruct((n, dout), jnp.float32),
    )(a, w, bias)


def _sc_gather(table, idx2d, F):
    nw = 32
    rpw = idx2d.shape[1]

    mesh = plsc.VectorSubcoreMesh(core_axis_name="c", subcore_axis_name="s")

    @functools.partial(
        pl.kernel,
        mesh=mesh,
        compiler_params=pltpu.CompilerParams(use_tc_tiling_on_sc=False),
        out_type=jax.ShapeDtypeStruct((E, F), jnp.float32),
        scratch_types=[
            pltpu.VMEM((rpw, 128), jnp.int32),
            pltpu.VMEM((128, F), jnp.float32),
            pltpu.VMEM((128, F), jnp.float32),
            pltpu.SemaphoreType.DMA,
            pltpu.SemaphoreType.DMA,
            pltpu.SemaphoreType.DMA,
            pltpu.SemaphoreType.DMA,
        ],
    )
    def gk(idx_hbm, table_hbm, out_hbm, idx_v, rows_a, rows_b, sia, sib,
           soa, sob):
        c = lax.axis_index("c")
        s = lax.axis_index("s")
        wid = s * 2 + c
        pltpu.sync_copy(idx_hbm.at[wid], idx_v)

        rows = (rows_a, rows_b)
        sin = (sia, sib)
        sout = (soa, sob)

        def start_in(j, b):
            return pltpu.async_copy(table_hbm.at[idx_v.at[j]], rows[b], sin[b])

        def start_out(j, b):
            return pltpu.async_copy(
                rows[b], out_hbm.at[pl.ds((wid * rpw + j) * 128, 128)],
                sout[b])

        hin = [None] * rpw
        hout = [None] * rpw
        hin[0] = start_in(0, 0)
        for j in range(rpw):
            b = j % 2
            if j + 1 < rpw:
                if j >= 1:
                    hout[j - 1].wait()
                hin[j + 1] = start_in(j + 1, (j + 1) % 2)
            hin[j].wait()
            hout[j] = start_out(j, b)
        if rpw >= 2:
            hout[rpw - 2].wait()
        hout[rpw - 1].wait()

    return gk(idx2d, table)


def _edge_body(w_ref, vg_ref, Wg_ref, W2_ref, b2_ref, g_ref, bt_ref, o_ref,
               st_ref):
    p = pl.program_id(0)
    j = pl.program_id(1)
    kk, r, dg = vg_ref.shape
    f1 = w_ref.shape[1]
    vgm = jnp.reshape(vg_ref[...], (kk * r, dg))
    if Wg_ref is not None:
        vgm = jnp.dot(vgm, Wg_ref[...], preferred_element_type=jnp.float32)
    h1 = jnp.reshape(vgm, (kk, r, f1)) + w_ref[...][None]

    @pl.when((p == 0) & (j == 0))
    def _():
        st_ref[...] = jnp.zeros_like(st_ref)

    @pl.when(p == 0)
    def _():
        h1m = jnp.reshape(h1, (kk * r, f1))
        both = jnp.concatenate([h1m, h1m * h1m], axis=1)
        ones = jnp.ones((8, kk * r), jnp.float32)
        s12 = jnp.dot(ones, both, preferred_element_type=jnp.float32)
        st_ref[...] += s12[0:1, :]
        o_ref[...] = jnp.zeros_like(o_ref)

    @pl.when(p == 1)
    def _():
        cnt = float(E)
        mu = st_ref[0:1, 0:f1] / cnt
        var = st_ref[0:1, f1:] / cnt - mu * mu
        sc = g_ref[...] * lax.rsqrt(var + 1e-5)
        hn = (h1 - mu[None]) * sc[None] + bt_ref[...][None]
        hr = jnp.maximum(hn, 0.0)
        h2 = jnp.dot(
            jnp.reshape(hr, (kk * r, f1)),
            W2_ref[...],
            preferred_element_type=jnp.float32,
        ) + b2_ref[...]
        f2 = h2.shape[1]
        o_ref[...] = jnp.max(jnp.reshape(h2, (kk, r, f2)), axis=0)


def _edge_stage(w, vg3, W2, b2, gamma, beta, Wg=None):
    F1 = w.shape[1]
    F2 = W2.shape[1]
    Dg = vg3.shape[2]
    nblk = 16
    R = N // nblk
    body = _edge_body if Wg is not None else (
        lambda w_r, vg_r, W2_r, b2_r, g_r, bt_r, o_r, st_r:
        _edge_body(w_r, vg_r, None, W2_r, b2_r, g_r, bt_r, o_r, st_r))
    in_specs = [
        pl.BlockSpec((R, F1), lambda p, j: (j, 0)),
        pl.BlockSpec((K, R, Dg), lambda p, j: (0, j, 0)),
    ]
    args = [w, vg3]
    if Wg is not None:
        in_specs.append(pl.BlockSpec((Dg, F1), lambda p, j: (0, 0)))
        args.append(Wg)
    in_specs += [
        pl.BlockSpec((F1, F2), lambda p, j: (0, 0)),
        pl.BlockSpec((1, F2), lambda p, j: (0, 0)),
        pl.BlockSpec((1, F1), lambda p, j: (0, 0)),
        pl.BlockSpec((1, F1), lambda p, j: (0, 0)),
    ]
    args += [W2, b2[None], gamma[None], beta[None]]
    return pl.pallas_call(
        body,
        grid=(2, nblk),
        in_specs=in_specs,
        out_specs=pl.BlockSpec((R, F2), lambda p, j: (j, 0)),
        out_shape=jax.ShapeDtypeStruct((N, F2), jnp.float32),
        scratch_shapes=[pltpu.VMEM((1, 2 * F1), jnp.float32)],
    )(*args)


def _edge_conv(feat, nbr, layers, raw_gather=False):
    (W1, b1, gamma, beta), (W2, b2) = layers
    d = feat.shape[1]
    Wt, Wb = W1[:d], W1[d:]
    F1 = W2.shape[0]
    idx = jnp.reshape(jnp.transpose(jnp.reshape(nbr, (N, K))),
                      (32, E // (32 * 128), 128))
    if raw_gather:
        w = _matmul(_pad_cols(feat), _pad_rows(Wt - Wb), b1[None])
        table = _pad_cols(feat, 16)
        dg = table.shape[1]
        vg = _sc_gather(table, idx, dg)
        vg3 = jnp.reshape(vg, (K, N, dg))
        return _edge_stage(w, vg3, W2, b2, gamma, beta,
                           Wg=_pad_rows(Wb, 16))
    Wd = jnp.concatenate([Wt - Wb, Wb], axis=1)
    bias = jnp.concatenate([b1, jnp.zeros_like(b1)])[None]
    uv = _matmul(_pad_cols(feat), _pad_rows(Wd), bias)
    w = uv[:, :F1]
    v = uv[:, F1:]
    vg = _sc_gather(v, idx, F1)
    vg3 = jnp.reshape(vg, (K, N, F1))
    return _edge_stage(w, vg3, W2, b2, gamma, beta)


def _fe_body(a_ref, W1_ref, b1_ref, g_ref, bt_ref, W2_ref, b2_ref,
             o_ref, st_ref, mx_ref):
    p = pl.program_id(0)
    j = pl.program_id(1)
    nblk = pl.num_programs(1)
    t = jnp.dot(a_ref[...], W1_ref[...],
                preferred_element_type=jnp.float32) + b1_ref[...]

    @pl.when((p == 0) & (j == 0))
    def _():
        st_ref[...] = jnp.zeros_like(st_ref)

    @pl.when(p == 0)
    def _():
        st_ref[0:1, :] += jnp.sum(t, axis=0)[None]
        st_ref[1:2, :] += jnp.sum(t * t, axis=0)[None]
        o_ref[...] = jnp.zeros_like(o_ref)

    @pl.when(p == 1)
    def _():
        cnt = float(N)
        mu = st_ref[0:1, :] / cnt
        var = st_ref[1:2, :] / cnt - mu * mu
        sc = g_ref[...] * lax.rsqrt(var + 1e-5)
        hr = jnp.maximum((t - mu) * sc + bt_ref[...], 0.0)
        x5 = jnp.dot(hr, W2_ref[...],
                     preferred_element_type=jnp.float32) + b2_ref[...]
        bm = jnp.max(x5, axis=0)[None]

        @pl.when(j == 0)
        def _():
            mx_ref[...] = jnp.full_like(mx_ref, -BIG)

        mx_ref[...] = jnp.maximum(mx_ref[...], bm)

        @pl.when(j == nblk - 1)
        def _():
            o_ref[...] = mx_ref[...]


def _feature_extract(l4p, fe):
    (W1, b1, gamma, beta), (W2, b2) = fe
    W1p = _pad_rows(W1)
    din = l4p.shape[1]
    H = W1.shape[1]
    nblk = 8
    R = N // nblk
    return pl.pallas_call(
        _fe_body,
        grid=(2, nblk),
        in_specs=[
            pl.BlockSpec((R, din), lambda p, j: (j, 0)),
            pl.BlockSpec((din, H), lambda p, j: (0, 0)),
            pl.BlockSpec((1, H), lambda p, j: (0, 0)),
            pl.BlockSpec((1, H), lambda p, j: (0, 0)),
            pl.BlockSpec((1, H), lambda p, j: (0, 0)),
            pl.BlockSpec((H, H), lambda p, j: (0, 0)),
            pl.BlockSpec((1, H), lambda p, j: (0, 0)),
        ],
        out_specs=pl.BlockSpec((1, H), lambda p, j: (0, 0)),
        out_shape=jax.ShapeDtypeStruct((1, H), jnp.float32),
        scratch_shapes=[
            pltpu.VMEM((2, H), jnp.float32),
            pltpu.VMEM((1, H), jnp.float32),
        ],
    )(l4p, W1p, b1[None], gamma[None], beta[None], W2, b2[None])


def _head_body(a_ref, gf_ref, W0t_ref, W0b_ref, b0_ref, W1_ref, b1_ref,
               W2_ref, b2_ref, W3_ref, b3_ref, o_ref):
    gbias = jnp.dot(gf_ref[...], W0b_ref[...],
                    preferred_element_type=jnp.float32) + b0_ref[...]
    h = jnp.dot(a_ref[...], W0t_ref[...],
                preferred_element_type=jnp.float32) + gbias
    h = jnp.maximum(h, 0.0)
    h = jnp.maximum(
        jnp.dot(h, W1_ref[...], preferred_element_type=jnp.float32)
        + b1_ref[...], 0.0)
    h = jnp.maximum(
        jnp.dot(h, W2_ref[...], preferred_element_type=jnp.float32)
        + b2_ref[...], 0.0)
    o = jnp.dot(h, W3_ref[...], preferred_element_type=jnp.float32) \
        + b3_ref[...]
    colmask = lax.broadcasted_iota(jnp.int32, o.shape, 1) < 50
    o = jnp.where(colmask, o, -BIG)
    m = jnp.max(o, axis=1, keepdims=True)
    z = jnp.sum(jnp.exp(o - m), axis=1, keepdims=True)
    o_ref[...] = o - m - jnp.log(z)


def _head(l4p, gf, head):
    (W0, b0), (W1, b1), (W2, b2), (W3, b3) = head
    din = l4p.shape[1]
    W0t = _pad_rows(W0[:326])
    W0b = W0[326:]
    W3p = _pad_cols(W3, 64)
    b3p = _pad_cols(b3[None], 64)
    blk = 512
    out = pl.pallas_call(
        _head_body,
        grid=(N // blk,),
        in_specs=[
            pl.BlockSpec((blk, din), lambda j: (j, 0)),
            pl.BlockSpec((1, 1024), lambda j: (0, 0)),
            pl.BlockSpec((din, 256), lambda j: (0, 0)),
            pl.BlockSpec((1024, 256), lambda j: (0, 0)),
            pl.BlockSpec((1, 256), lambda j: (0, 0)),
            pl.BlockSpec((256, 256), lambda j: (0, 0)),
            pl.BlockSpec((1, 256), lambda j: (0, 0)),
            pl.BlockSpec((256, 128), lambda j: (0, 0)),
            pl.BlockSpec((1, 128), lambda j: (0, 0)),
            pl.BlockSpec((128, 64), lambda j: (0, 0)),
            pl.BlockSpec((1, 64), lambda j: (0, 0)),
        ],
        out_specs=pl.BlockSpec((blk, 64), lambda j: (j, 0)),
        out_shape=jax.ShapeDtypeStruct((N, 64), jnp.float32),
    )(l4p, gf, W0t, W0b, b0[None], W1, b1[None], W2, b2[None], W3p, b3p)
    return out[:, :50]


def kernel(x, pos, batch, params):
    x0 = jnp.concatenate([x, pos], axis=1)

    nbr = _knn(_pad_cols(x0).reshape(B, P, 8))
    x1 = _edge_conv(x0, nbr, params["ec1"], raw_gather=True)

    nbr = _knn(x1.reshape(B, P, 64))
    l1 = jnp.concatenate([x0, x1], axis=1)
    x2 = _edge_conv(l1, nbr, params["ec2"])

    nbr = _knn(x2.reshape(B, P, 64))
    l2 = jnp.concatenate([x0, x1, x2], axis=1)
    x3 = _edge_conv(l2, nbr, params["ec3"])

    l3 = jnp.concatenate([x0, x1, x2, x3], axis=1)
    x4 = _edge_conv(l3, nbr, params["ec4"])

    l4 = jnp.concatenate([x0, x1, x2, x3, x4], axis=1)
    l4p = _pad_cols(l4)
    gf = _feature_extract(l4p, params["fe"])
    return _head(l4p, gf, params["head"])
